# speedup vs baseline: 1.0083x; 1.0083x over previous
import math

import jax
import jax.numpy as jnp
from jax.experimental import pallas as pl
from jax.experimental.pallas import tpu as pltpu

_N_DELTA = 4
_N_THETA = 8
_N_GAMMA = 16
_N_OSC = _N_DELTA + _N_THETA + _N_GAMMA
_N_STEPS = 5
_THRESH = 0.3
_EPS = 1e-6
_TWO_PI = 2.0 * math.pi

_BM = 512


def _sim_body(ra_ref, ia_ref, rb_ref, ib_ref, sim_ref, ms_ref, mi_ref):
    ra = ra_ref[...]
    ia = ia_ref[...]
    rb = rb_ref[...]
    ib = ib_ref[...]
    dn = (((1,), (1,)), ((), ()))
    sim = jax.lax.dot_general(ra, rb, dn, preferred_element_type=jnp.float32)
    sim = sim + jax.lax.dot_general(ia, ib, dn, preferred_element_type=jnp.float32)
    sim_ref[...] = sim
    m = jnp.max(sim, axis=1, keepdims=True)
    ms_ref[...] = m
    n_t1 = sim.shape[1]
    col = jax.lax.broadcasted_iota(jnp.int32, sim.shape, 1)
    cand = jnp.where(sim == m, col, n_t1)
    mi_ref[...] = jnp.min(cand, axis=1, keepdims=True)


def _sim_call(ra, ia, rb, ib):
    n_t, k = ra.shape
    n_t1 = rb.shape[0]
    grid = (pl.cdiv(n_t, _BM),)
    return pl.pallas_call(
        _sim_body,
        grid=grid,
        in_specs=[
            pl.BlockSpec((_BM, k), lambda i: (i, 0)),
            pl.BlockSpec((_BM, k), lambda i: (i, 0)),
            pl.BlockSpec((n_t1, k), lambda i: (0, 0)),
            pl.BlockSpec((n_t1, k), lambda i: (0, 0)),
        ],
        out_specs=[
            pl.BlockSpec((_BM, n_t1), lambda i: (i, 0)),
            pl.BlockSpec((_BM, 1), lambda i: (i, 0)),
            pl.BlockSpec((_BM, 1), lambda i: (i, 0)),
        ],
        out_shape=[
            jax.ShapeDtypeStruct((n_t, n_t1), jnp.float32),
            jax.ShapeDtypeStruct((n_t, 1), jnp.float32),
            jax.ShapeDtypeStruct((n_t, 1), jnp.int32),
        ],
    )(ra, ia, rb, ib)


def kernel(detections_t, detections_t1, Wp1, bp1, Wp2, bp2, Wa1, ba1, Wa2, ba2):
    freqs = jnp.concatenate([
        jnp.full((_N_DELTA,), 2.0, jnp.float32),
        jnp.full((_N_THETA,), 6.0, jnp.float32),
        jnp.full((_N_GAMMA,), 40.0, jnp.float32),
    ])

    def encode_phase(d):
        h = jax.nn.relu(d @ Wp1 + bp1)
        return (h @ Wp2 + bp2) % _TWO_PI

    phase_t = encode_phase(detections_t)
    phase_t1 = encode_phase(detections_t1)

    dt = 0.01
    for _ in range(_N_STEPS):
        phase_t = (phase_t + _TWO_PI * freqs * dt) % _TWO_PI

    def unit(p):
        re = jnp.cos(p)
        im = jnp.sin(p)
        nrm = jnp.sqrt(jnp.sum(re * re + im * im, axis=-1, keepdims=True)) + _EPS
        return re / nrm, im / nrm

    ra, ia = unit(phase_t)
    rb, ib = unit(phase_t1)

    sim, ms, mi = _sim_call(ra, ia, rb, ib)
    max_sims = ms[:, 0]
    max_idxs = mi[:, 0]

    n_t = detections_t.shape[0]
    n_t1 = detections_t1.shape[0]
    order = jnp.argsort(-max_sims)
    matches0 = jnp.full((n_t,), -1, jnp.int32)
    used0 = jnp.zeros((n_t1,), bool)

    def step(carry, idx):
        matches, used = carry
        j = max_idxs[idx]
        ok = jnp.logical_and(jnp.logical_not(used[j]), max_sims[idx] > _THRESH)
        matches = matches.at[idx].set(jnp.where(ok, j.astype(jnp.int32), matches[idx]))
        used = used.at[j].set(jnp.logical_or(used[j], ok))
        return (matches, used), None

    (matches, used), _ = jax.lax.scan(step, (matches0, used0), order)
    return matches, sim


# speedup vs baseline: 356.3515x; 353.4036x over previous
import functools
import math

import jax
import jax.numpy as jnp
from jax import lax
from jax.experimental import pallas as pl
from jax.experimental.pallas import tpu as pltpu
from jax.experimental.pallas import tpu_sc as plsc

_N_DELTA = 4
_N_THETA = 8
_N_GAMMA = 16
_N_STEPS = 5
_THRESH = 0.3
_EPS = 1e-6
_TWO_PI = 2.0 * math.pi

_BM = 512
_N_PAD = 5120
_LANES = 16


def _sim_body(ra_ref, ia_ref, rb_ref, ib_ref, sim_ref, ms_ref, mi_ref):
    ra = ra_ref[...]
    ia = ia_ref[...]
    rb = rb_ref[...]
    ib = ib_ref[...]
    dn = (((1,), (1,)), ((), ()))
    sim = jax.lax.dot_general(ra, rb, dn, preferred_element_type=jnp.float32)
    sim = sim + jax.lax.dot_general(ia, ib, dn, preferred_element_type=jnp.float32)
    sim_ref[...] = sim
    m = jnp.max(sim, axis=1, keepdims=True)
    ms_ref[...] = m
    n_t1 = sim.shape[1]
    col = jax.lax.broadcasted_iota(jnp.int32, sim.shape, 1)
    cand = jnp.where(sim == m, col, n_t1)
    mi_ref[...] = jnp.min(cand, axis=1, keepdims=True)


def _sim_call(ra, ia, rb, ib):
    n_t, k = ra.shape
    n_t1 = rb.shape[0]
    grid = (pl.cdiv(n_t, _BM),)
    return pl.pallas_call(
        _sim_body,
        grid=grid,
        in_specs=[
            pl.BlockSpec((_BM, k), lambda i: (i, 0)),
            pl.BlockSpec((_BM, k), lambda i: (i, 0)),
            pl.BlockSpec((n_t1, k), lambda i: (0, 0)),
            pl.BlockSpec((n_t1, k), lambda i: (0, 0)),
        ],
        out_specs=[
            pl.BlockSpec((_BM, n_t1), lambda i: (i, 0)),
            pl.BlockSpec((_BM, 1), lambda i: (i, 0)),
            pl.BlockSpec((_BM, 1), lambda i: (i, 0)),
        ],
        out_shape=[
            jax.ShapeDtypeStruct((n_t, n_t1), jnp.float32),
            jax.ShapeDtypeStruct((n_t, 1), jnp.float32),
            jax.ShapeDtypeStruct((n_t, 1), jnp.int32),
        ],
    )(ra, ia, rb, ib)


def _rank_body(ms_ref, msr_ref, rank_ref, thr_ref):
    pid = pl.program_id(0)
    msb = ms_ref[...]
    msr = msr_ref[...]
    gt = (msr > msb).astype(jnp.int32)
    col = jax.lax.broadcasted_iota(jnp.int32, gt.shape, 1)
    row = pid * _BM + jax.lax.broadcasted_iota(jnp.int32, gt.shape, 0)
    tie = jnp.logical_and(msr == msb, col < row).astype(jnp.int32)
    rank_ref[...] = jnp.sum(gt + tie, axis=1, keepdims=True)
    thr_ref[...] = (msb > _THRESH).astype(jnp.int32)


def _rank_call(ms, msr):
    n_t = ms.shape[0]
    n_t1 = msr.shape[1]
    grid = (pl.cdiv(n_t, _BM),)
    return pl.pallas_call(
        _rank_body,
        grid=grid,
        in_specs=[
            pl.BlockSpec((_BM, 1), lambda i: (i, 0)),
            pl.BlockSpec((1, n_t1), lambda i: (0, 0)),
        ],
        out_specs=[
            pl.BlockSpec((_BM, 1), lambda i: (i, 0)),
            pl.BlockSpec((_BM, 1), lambda i: (i, 0)),
        ],
        out_shape=[
            jax.ShapeDtypeStruct((n_t, 1), jnp.int32),
            jax.ShapeDtypeStruct((n_t, 1), jnp.int32),
        ],
    )(ms, msr)


def _greedy_body(n_t, n_t1, rank_hbm, mi_hbm, thr_hbm, out_hbm,
                 rank_v, mi_v, thr_v, order_v, used_v, match_v, jwin_v):
    c = lax.axis_index("c")
    s = lax.axis_index("s")

    @pl.when(jnp.logical_and(c == 0, s == 0))
    def _():
        pltpu.sync_copy(rank_hbm, rank_v)
        pltpu.sync_copy(mi_hbm, mi_v)
        pltpu.sync_copy(thr_hbm, thr_v)

        lane = jax.lax.iota(jnp.int32, _LANES)
        ones = jnp.ones((_LANES,), jnp.int32)
        nwin = _N_PAD // _LANES

        def init_body(k, carry):
            base = k * _LANES
            idxr = rank_v[pl.ds(base, _LANES)]
            plsc.store_scatter(order_v, [idxr], lane + base)
            used_v[pl.ds(base, _LANES)] = jnp.zeros((_LANES,), jnp.int32)
            return carry

        lax.fori_loop(0, nwin, init_body, 0)

        def win_body(k, carry):
            idxv = order_v[pl.ds(k * _LANES, _LANES)]
            jv = plsc.load_gather(mi_v, [idxv])
            thrv = plsc.load_gather(thr_v, [idxv])
            usedv = plsc.load_gather(used_v, [jv])
            jwin_v[...] = jv
            seen = jnp.zeros((_LANES,), jnp.bool_)
            for sh in range(1, _LANES):
                perm = (lane + (_LANES - sh)) & (_LANES - 1)
                rolled = plsc.load_gather(jwin_v, [perm])
                dup = jnp.logical_and(rolled == jv, lane >= sh)
                seen = jnp.logical_or(seen, dup)
            claim = jnp.logical_and(
                jnp.logical_and(thrv == 1, usedv == 0),
                jnp.logical_not(seen))
            plsc.store_scatter(used_v, [jv], ones, mask=claim)
            plsc.store_scatter(match_v, [idxv], jnp.where(claim, jv, -1))
            return carry

        lax.fori_loop(0, nwin, win_body, 0)

        pltpu.sync_copy(match_v, out_hbm)


def _greedy_call(rank_p, mi_p, thr_p, n_t, n_t1):
    mesh = plsc.VectorSubcoreMesh(core_axis_name="c", subcore_axis_name="s")
    return pl.kernel(
        functools.partial(_greedy_body, n_t, n_t1),
        out_type=jax.ShapeDtypeStruct((_N_PAD,), jnp.int32),
        mesh=mesh,
        compiler_params=pltpu.CompilerParams(needs_layout_passes=False),
        scratch_types=[
            pltpu.VMEM((_N_PAD,), jnp.int32),
            pltpu.VMEM((_N_PAD,), jnp.int32),
            pltpu.VMEM((_N_PAD,), jnp.int32),
            pltpu.VMEM((_N_PAD,), jnp.int32),
            pltpu.VMEM((_N_PAD,), jnp.int32),
            pltpu.VMEM((_N_PAD,), jnp.int32),
            pltpu.VMEM((_LANES,), jnp.int32),
        ],
    )(rank_p, mi_p, thr_p)


def kernel(detections_t, detections_t1, Wp1, bp1, Wp2, bp2, Wa1, ba1, Wa2, ba2):
    freqs = jnp.concatenate([
        jnp.full((_N_DELTA,), 2.0, jnp.float32),
        jnp.full((_N_THETA,), 6.0, jnp.float32),
        jnp.full((_N_GAMMA,), 40.0, jnp.float32),
    ])

    def encode_phase(d):
        h = jax.nn.relu(d @ Wp1 + bp1)
        return (h @ Wp2 + bp2) % _TWO_PI

    phase_t = encode_phase(detections_t)
    phase_t1 = encode_phase(detections_t1)

    dt = 0.01
    for _ in range(_N_STEPS):
        phase_t = (phase_t + _TWO_PI * freqs * dt) % _TWO_PI

    def unit(p):
        re = jnp.cos(p)
        im = jnp.sin(p)
        nrm = jnp.sqrt(jnp.sum(re * re + im * im, axis=-1, keepdims=True)) + _EPS
        return re / nrm, im / nrm

    ra, ia = unit(phase_t)
    rb, ib = unit(phase_t1)

    sim, ms, mi = _sim_call(ra, ia, rb, ib)

    n_t = detections_t.shape[0]
    n_t1 = detections_t1.shape[0]

    rank, thr = _rank_call(ms, ms.reshape(1, n_t))

    pad = _N_PAD - n_t
    rank_p = jnp.concatenate([rank[:, 0], jnp.arange(n_t, _N_PAD, dtype=jnp.int32)])
    mi_p = jnp.concatenate([mi[:, 0], jnp.zeros((pad,), jnp.int32)])
    thr_p = jnp.concatenate([thr[:, 0], jnp.zeros((pad,), jnp.int32)])

    matches = _greedy_call(rank_p, mi_p, thr_p, n_t, n_t1)[:n_t]
    return matches, sim
